# baseline (speedup 1.0000x reference)
import jax
import jax.numpy as jnp
from jax import lax
from jax.experimental import pallas as pl
from jax.experimental.pallas import tpu as pltpu

K = 16


def _topk_kernel(act_hbm, sens_hbm, vals_ref, idxs_ref,
                 abuf, sbuf, asem, ssem):
    b = pl.program_id(0)
    nb = pl.num_programs(0)
    slot = lax.rem(b, 2)
    nslot = lax.rem(b + 1, 2)

    @pl.when(b == 0)
    def _():
        pltpu.make_async_copy(act_hbm.at[0], abuf.at[0], asem.at[0]).start()
        pltpu.make_async_copy(sens_hbm.at[0], sbuf.at[0], ssem.at[0]).start()

    @pl.when(b + 1 < nb)
    def _():
        pltpu.make_async_copy(
            act_hbm.at[b + 1], abuf.at[nslot], asem.at[nslot]).start()
        pltpu.make_async_copy(
            sens_hbm.at[b + 1], sbuf.at[nslot], ssem.at[nslot]).start()

    pltpu.make_async_copy(act_hbm.at[b], abuf.at[slot], asem.at[slot]).wait()
    pltpu.make_async_copy(sens_hbm.at[b], sbuf.at[slot], ssem.at[slot]).wait()

    act = abuf[slot]
    sens = sbuf[slot]
    A, L = act.shape
    S = sens.shape[0]

    xn = jnp.sqrt(jnp.sum(act * act, axis=1))
    yn = jnp.sqrt(jnp.sum(sens * sens, axis=1))

    num_t = lax.dot_general(
        sens, act,
        dimension_numbers=(((1,), (1,)), ((), ())),
        precision=lax.Precision.DEFAULT,
        preferred_element_type=jnp.float32,
    )
    sim = num_t / (yn[:, None] * xn[None, :])

    score = sim * sim
    iota = lax.broadcasted_iota(jnp.int32, (S, A), 0)
    sign = lax.shift_right_logical(
        lax.bitcast_convert_type(sim, jnp.int32), 31)
    fpacked = (iota * 2 + sign).astype(jnp.float32)
    bigf = jnp.float32(1e9)

    R = 32
    NS = S // R
    sc_slabs = [score[r * R:(r + 1) * R] for r in range(NS)]
    fp_slabs = [fpacked[r * R:(r + 1) * R] for r in range(NS)]

    idxp_f = None
    for j in range(K):
        acc = None
        for r in range(NS):
            s = sc_slabs[r]
            if idxp_f is not None:
                s = jnp.where(fp_slabs[r] == idxp_f[None, :], -1.0, s)
                sc_slabs[r] = s
            acc = s if acc is None else jnp.maximum(acc, s)
        m = jnp.max(acc, axis=0)

        acc2 = None
        for r in range(NS):
            c = jnp.where(sc_slabs[r] >= m[None, :], fp_slabs[r], bigf)
            acc2 = c if acc2 is None else jnp.minimum(acc2, c)
        idxp_f = jnp.min(acc2, axis=0)

        idxp = idxp_f.astype(jnp.int32)
        rt = jnp.sqrt(m)
        val = jnp.where((idxp & 1) == 1, -rt, rt)
        vals_ref[0, j, :] = val
        idxs_ref[0, j, :] = lax.shift_right_logical(idxp, 1)


@jax.jit
def kernel(x_actuators, x_sensors):
    B, A, L = x_actuators.shape
    S = x_sensors.shape[1]
    k = K

    vals_t, idxs_t = pl.pallas_call(
        _topk_kernel,
        grid=(B,),
        compiler_params=pltpu.CompilerParams(
            dimension_semantics=("arbitrary",),
        ),
        in_specs=[
            pl.BlockSpec(memory_space=pl.ANY),
            pl.BlockSpec(memory_space=pl.ANY),
        ],
        scratch_shapes=[
            pltpu.VMEM((2, A, L), jnp.float32),
            pltpu.VMEM((2, S, L), jnp.float32),
            pltpu.SemaphoreType.DMA((2,)),
            pltpu.SemaphoreType.DMA((2,)),
        ],
        out_specs=[
            pl.BlockSpec((1, k, A), lambda b: (b, 0, 0)),
            pl.BlockSpec((1, k, A), lambda b: (b, 0, 0)),
        ],
        out_shape=[
            jax.ShapeDtypeStruct((B, k, A), jnp.float32),
            jax.ShapeDtypeStruct((B, k, A), jnp.int32),
        ],
    )(x_actuators, x_sensors)

    target_nodes = jnp.swapaxes(idxs_t, 1, 2).reshape(B, A * k)
    source_nodes = jnp.tile(jnp.repeat(jnp.arange(A), k)[None, :], (B, 1))
    edges = jnp.stack([source_nodes, target_nodes], axis=1)
    weights = jnp.swapaxes(vals_t, 1, 2).reshape(B, A * k)
    return edges, weights


# speedup vs baseline: 1.0104x; 1.0104x over previous
import jax
import jax.numpy as jnp
from jax import lax
from jax.experimental import pallas as pl
from jax.experimental.pallas import tpu as pltpu

K = 16


def _topk_kernel(act_ref, sens_ref, vals_ref, idxs_ref):
    act = act_ref[0]
    sens = sens_ref[0]
    A, L = act.shape
    S = sens.shape[0]

    xn = jnp.sqrt(jnp.sum(act * act, axis=1))
    yn = jnp.sqrt(jnp.sum(sens * sens, axis=1))

    num_t = lax.dot_general(
        sens, act,
        dimension_numbers=(((1,), (1,)), ((), ())),
        precision=lax.Precision.DEFAULT,
        preferred_element_type=jnp.float32,
    )
    sim = num_t / (yn[:, None] * xn[None, :])

    score = sim * sim
    iota = lax.broadcasted_iota(jnp.int32, (S, A), 0)
    sign = lax.shift_right_logical(
        lax.bitcast_convert_type(sim, jnp.int32), 31)
    fpacked = (iota * 2 + sign).astype(jnp.float32)
    bigf = jnp.float32(1e9)

    R = 16
    NS = S // R
    sc_slabs = [score[r * R:(r + 1) * R] for r in range(NS)]
    fp_slabs = [fpacked[r * R:(r + 1) * R] for r in range(NS)]

    idxp_f = None
    for j in range(K):
        acc = None
        for r in range(NS):
            s = sc_slabs[r]
            if idxp_f is not None:
                s = jnp.where(fp_slabs[r] == idxp_f[None, :], -1.0, s)
                sc_slabs[r] = s
            acc = s if acc is None else jnp.maximum(acc, s)
        m = jnp.max(acc, axis=0)

        acc2 = None
        for r in range(NS):
            c = jnp.where(sc_slabs[r] >= m[None, :], fp_slabs[r], bigf)
            acc2 = c if acc2 is None else jnp.minimum(acc2, c)
        idxp_f = jnp.min(acc2, axis=0)

        idxp = idxp_f.astype(jnp.int32)
        rt = jnp.sqrt(m)
        val = jnp.where((idxp & 1) == 1, -rt, rt)
        vals_ref[0, j, :] = val
        idxs_ref[0, j, :] = lax.shift_right_logical(idxp, 1)


@jax.jit
def kernel(x_actuators, x_sensors):
    B, A, L = x_actuators.shape
    S = x_sensors.shape[1]
    k = K

    vals_t, idxs_t = pl.pallas_call(
        _topk_kernel,
        grid=(B,),
        compiler_params=pltpu.CompilerParams(
            dimension_semantics=("arbitrary",),
        ),
        in_specs=[
            pl.BlockSpec((1, A, L), lambda b: (b, 0, 0)),
            pl.BlockSpec((1, S, L), lambda b: (b, 0, 0)),
        ],
        out_specs=[
            pl.BlockSpec((1, k, A), lambda b: (b, 0, 0)),
            pl.BlockSpec((1, k, A), lambda b: (b, 0, 0)),
        ],
        out_shape=[
            jax.ShapeDtypeStruct((B, k, A), jnp.float32),
            jax.ShapeDtypeStruct((B, k, A), jnp.int32),
        ],
    )(x_actuators, x_sensors)

    target_nodes = jnp.swapaxes(idxs_t, 1, 2).reshape(B, A * k)
    source_nodes = jnp.tile(jnp.repeat(jnp.arange(A), k)[None, :], (B, 1))
    edges = jnp.stack([source_nodes, target_nodes], axis=1)
    weights = jnp.swapaxes(vals_t, 1, 2).reshape(B, A * k)
    return edges, weights
